# Initial kernel scaffold; baseline (speedup 1.0000x reference)
import jax
import jax.numpy as jnp
from jax.experimental import pallas as pl


def kernel(x, mask, table):
    raise NotImplementedError("write your pallas kernel here")


# speedup vs baseline: 1.1495x; 1.1495x over previous
import jax
import jax.numpy as jnp
from jax.experimental import pallas as pl

_NUM_EMB = 512
_EMB_DIM = 64
_BETA = 0.25
_TB = 128


def _fold_half(xm, tT, base):
    def F(s):
        f = None
        for j in range(4):
            e = base + 8 * j + s
            u = jnp.abs(xm[:, e:e + 1] - tT[e:e + 1, :])
            f = u if f is None else f + u
        return f

    g04 = F(0) + F(4)
    g26 = F(2) + F(6)
    h0 = g04 + g26
    g15 = F(1) + F(5)
    g37 = F(3) + F(7)
    return h0 + (g15 + g37)


def _block(x_ref, m_ref, t_ref, tT_ref, o_ref, loss_ref):
    xb = x_ref[...]
    mb = m_ref[...]
    xm = jnp.where(mb > 0, 0.0, xb)
    tT = tT_ref[...]

    d = _fold_half(xm, tT, 0) + _fold_half(xm, tT, 32)

    dmin = jnp.min(d, axis=1, keepdims=True)
    iota = jax.lax.broadcasted_iota(jnp.int32, d.shape, 1)
    idx = jnp.min(jnp.where(d == dmin, iota, _NUM_EMB), axis=1, keepdims=True)
    onehot = (iota == idx).astype(jnp.float32)
    zq = jax.lax.dot_general(
        onehot, t_ref[...], (((1,), (0,)), ((), ())),
        precision=jax.lax.Precision.HIGHEST,
        preferred_element_type=jnp.float32)
    zq = jnp.where(mb > 0, 0.0, zq)

    o_ref[...] = xm + (zq - xm)
    diff = xm - zq
    part = jnp.sum(diff * diff).reshape(1, 1)

    @pl.when(pl.program_id(0) == 0)
    def _init():
        loss_ref[...] = part

    @pl.when(pl.program_id(0) != 0)
    def _acc():
        loss_ref[...] += part


def kernel(x, mask, table):
    B, T, E = x.shape
    N = B * T
    xf = x.reshape(N, E)
    mf = mask.reshape(N, 1).astype(jnp.float32)
    tT = jnp.swapaxes(table, 0, 1)

    zf, losssum = pl.pallas_call(
        _block,
        grid=(N // _TB,),
        in_specs=[
            pl.BlockSpec((_TB, E), lambda i: (i, 0)),
            pl.BlockSpec((_TB, 1), lambda i: (i, 0)),
            pl.BlockSpec((_NUM_EMB, E), lambda i: (0, 0)),
            pl.BlockSpec((E, _NUM_EMB), lambda i: (0, 0)),
        ],
        out_specs=[
            pl.BlockSpec((_TB, E), lambda i: (i, 0)),
            pl.BlockSpec((1, 1), lambda i: (0, 0)),
        ],
        out_shape=[
            jax.ShapeDtypeStruct((N, E), jnp.float32),
            jax.ShapeDtypeStruct((1, 1), jnp.float32),
        ],
    )(xf, mf, table, tT)

    emb = losssum[0, 0] / (N * E)
    return (zf.reshape(B, T, E), emb, _BETA * emb)
